# baseline (speedup 1.0000x reference)
import dataclasses
import functools

import jax
import jax.numpy as jnp
from jax import lax
from jax.experimental import pallas as pl
from jax.experimental.pallas import tpu as pltpu
from jax.experimental.pallas import tpu_sc as plsc

N = 10000
E = 320000
D = 128
DH = D // 2

NC = 2
NS = 16
NW = NC * NS
EPW = E // NW
EPT = E // NS
CHUNK = 100
NCHUNK = EPT // CHUNK
NBUF = 4
NPAD = 10240
L = 16
ZSTRIDE = 624
ZROWS = 640

_mesh = plsc.VectorSubcoreMesh(core_axis_name="c", subcore_axis_name="s")

_sc_params = pltpu.CompilerParams()
for _field, _val in (("needs_layout_passes", False),
                     ("use_tc_tiling_on_sc", False)):
    if _field in pltpu.CompilerParams.__dataclass_fields__:
        _sc_params = dataclasses.replace(_sc_params, **{_field: _val})


@functools.partial(
    pl.kernel,
    mesh=_mesh,
    compiler_params=_sc_params,
    out_type=jax.ShapeDtypeStruct((NW, NPAD), jnp.float32),
    scratch_types=[
        pltpu.VMEM((EPW,), jnp.int32),
        pltpu.VMEM((NPAD,), jnp.float32),
    ],
)
def _deg_kernel(src_hbm, degp_hbm, src_v, hist_v):
    cid = lax.axis_index("c")
    sid = lax.axis_index("s")
    wid = sid * NC + cid
    pltpu.sync_copy(src_hbm.at[pl.ds(wid * EPW, EPW)], src_v)

    zeros = jnp.zeros((L,), jnp.float32)
    ones = jnp.ones((L,), jnp.float32)

    @pl.loop(0, NPAD // L)
    def _(i):
        hist_v[pl.ds(i * L, L)] = zeros

    @pl.loop(0, EPW // L)
    def _(i):
        idx = src_v[pl.ds(i * L, L)]
        plsc.addupdate_scatter(hist_v, [idx], ones)

    pltpu.sync_copy(hist_v, degp_hbm.at[wid])


@functools.partial(
    pl.kernel,
    mesh=_mesh,
    compiler_params=_sc_params,
    out_type=jax.ShapeDtypeStruct((E,), jnp.float32),
    scratch_types=[
        pltpu.VMEM((EPW,), jnp.int32),
        pltpu.VMEM((EPW,), jnp.float32),
        pltpu.VMEM((NPAD,), jnp.float32),
    ],
)
def _c_kernel(src_hbm, w_hbm, inv_hbm, c_hbm, src_v, c_v, inv_v):
    cid = lax.axis_index("c")
    sid = lax.axis_index("s")
    wid = sid * NC + cid
    pltpu.sync_copy(src_hbm.at[pl.ds(wid * EPW, EPW)], src_v)
    pltpu.sync_copy(w_hbm.at[pl.ds(wid * EPW, EPW)], c_v)
    pltpu.sync_copy(inv_hbm, inv_v)

    @pl.loop(0, EPW // L)
    def _(i):
        s16 = src_v[pl.ds(i * L, L)]
        iv = plsc.load_gather(inv_v, [s16])
        c_v[pl.ds(i * L, L)] = c_v[pl.ds(i * L, L)] * iv

    pltpu.sync_copy(c_v, c_hbm.at[pl.ds(wid * EPW, EPW)])


@functools.partial(
    pl.kernel,
    mesh=_mesh,
    compiler_params=_sc_params,
    out_type=jax.ShapeDtypeStruct((NC, N, DH), jnp.float32),
    scratch_types=[
        pltpu.VMEM((NCHUNK, CHUNK), jnp.int32),
        pltpu.VMEM((NCHUNK, CHUNK), jnp.int32),
        pltpu.VMEM((EPT,), jnp.float32),
        pltpu.VMEM((NBUF, CHUNK, DH // 2), jnp.int32),
        pltpu.VMEM((CHUNK, DH), jnp.float32),
        pltpu.SemaphoreType.DMA,
        pltpu.SemaphoreType.DMA,
        pltpu.SemaphoreType.DMA,
        pltpu.SemaphoreType.DMA,
        pltpu.SemaphoreType.DMA,
        pltpu.VMEM_SHARED((N, DH), jnp.float32),
    ],
)
def _mp_kernel(table_hbm, src2_hbm, dst2_hbm, c_hbm,
               acc_hbm, src2_v, dst2_v, c_v, gbuf_v, sbuf_v,
               gsem0, gsem1, gsem2, gsem3, ssem, acc_sh):
    cid = lax.axis_index("c")
    sid = lax.axis_index("s")

    pltpu.sync_copy(src2_hbm.at[sid], src2_v)
    pltpu.sync_copy(dst2_hbm.at[sid], dst2_v)
    pltpu.sync_copy(c_hbm.at[pl.ds(sid * EPT, EPT)], c_v)

    zeros = jnp.zeros((L,), jnp.float32)

    @pl.loop(0, CHUNK)
    def _(e):
        for k in range(DH // L):
            sbuf_v[e, pl.ds(k * L, L)] = zeros

    base = sid * ZSTRIDE
    for r in range(ZROWS // CHUNK):
        pltpu.sync_copy(sbuf_v, acc_sh.at[pl.ds(base + r * CHUNK, CHUNK)])
    if ZROWS % CHUNK:
        pltpu.sync_copy(sbuf_v.at[pl.ds(0, ZROWS % CHUNK)],
                        acc_sh.at[pl.ds(base + (ZROWS // CHUNK) * CHUNK,
                                        ZROWS % CHUNK)])
    plsc.subcore_barrier()

    gsems = (gsem0, gsem1, gsem2, gsem3)

    def _gather(b, j):
        return pltpu.make_async_copy(table_hbm.at[cid].at[src2_v.at[j]],
                                     gbuf_v.at[b], gsems[b])

    def _scatter(b, j):
        return pltpu.async_copy(sbuf_v, acc_sh.at[dst2_v.at[j]],
                                ssem, add=True)

    mask_hi = jnp.full((L,), -65536, jnp.int32)

    def _scale(b, j):
        @pl.loop(0, CHUNK)
        def _(e):
            cvec = plsc.load_gather(
                c_v, [jnp.full((L,), j * CHUNK + e, jnp.int32)])
            for g in range(DH // 32):
                wvec = gbuf_v[b, e, pl.ds(g * L, L)]
                lo = plsc.bitcast(wvec << 16, jnp.float32)
                hi = plsc.bitcast(wvec & mask_hi, jnp.float32)
                sbuf_v[e, pl.ds(g * 32, L)] = lo * cvec
                sbuf_v[e, pl.ds(g * 32 + L, L)] = hi * cvec

    for b in range(NBUF):
        _gather(b, b).start()

    @pl.loop(0, NCHUNK // NBUF)
    def _(t):
        for b in range(NBUF):
            j = NBUF * t + b
            _gather(b, j).wait()
            _scale(b, j)
            _scatter(b, j).wait()

            @pl.when(j + NBUF < NCHUNK)
            def _():
                _gather(b, j + NBUF).start()

    plsc.subcore_barrier()
    pltpu.sync_copy(acc_sh.at[pl.ds(base, ZROWS)],
                    acc_hbm.at[cid, pl.ds(base, ZROWS)])


def _inv_body(degp_ref, inv_ref):
    deg = jnp.sum(degp_ref[...], axis=0, keepdims=True)
    inv_ref[...] = 1.0 / jnp.maximum(deg, 1.0)


def _tc_inv(degp):
    return pl.pallas_call(
        _inv_body,
        out_shape=jax.ShapeDtypeStruct((1, NPAD), jnp.float32),
    )(degp)


def _bf16_bits(t):
    b = jax.lax.bitcast_convert_type(t, jnp.int32)
    rnd = 0x7FFF + ((b >> 16) & 1)
    return (b + rnd) >> 16


def _pack_half(t):
    t4 = t.reshape(t.shape[0], DH // 32, 2, L)
    lo = _bf16_bits(t4[:, :, 0, :]) & 0xFFFF
    hi = _bf16_bits(t4[:, :, 1, :]) << 16
    return (lo | hi).reshape(t.shape[0], DH // 2)


def _mm_body(x_ref, w_ref, o_ref):
    x = x_ref[...]
    o_ref[0] = _pack_half(
        jnp.dot(x, w_ref[:, :DH], preferred_element_type=jnp.float32))
    o_ref[1] = _pack_half(
        jnp.dot(x, w_ref[:, DH:], preferred_element_type=jnp.float32))


def _mid_body(a_ref, b_ref, w_ref, o_ref):
    h = jnp.concatenate([a_ref[0], a_ref[1]], axis=1) + b_ref[...]
    h = jnp.maximum(h, 0.0)
    o_ref[0] = _pack_half(
        jnp.dot(h, w_ref[:, :DH], preferred_element_type=jnp.float32))
    o_ref[1] = _pack_half(
        jnp.dot(h, w_ref[:, DH:], preferred_element_type=jnp.float32))


def _last_body(a_ref, b_ref, h_ref, m_ref):
    i = pl.program_id(0)
    h = jnp.concatenate([a_ref[0], a_ref[1]], axis=1) + b_ref[...]
    h_ref[...] = h

    @pl.when(i == 0)
    def _():
        m_ref[...] = jnp.zeros_like(m_ref)

    m_ref[...] += jnp.sum(h, axis=0, keepdims=True)


_half_t = jax.ShapeDtypeStruct((NC, N, DH // 2), jnp.int32)
TB = 2000
_ablk = pl.BlockSpec((NC, TB, DH), lambda i: (0, i, 0))
_oblk = pl.BlockSpec((NC, TB, DH // 2), lambda i: (0, i, 0))


def _tc_mm(x, w):
    return pl.pallas_call(
        _mm_body,
        grid=(N // TB,),
        in_specs=[pl.BlockSpec((TB, D), lambda i: (i, 0)),
                  pl.BlockSpec((D, D), lambda i: (0, 0))],
        out_specs=_oblk,
        out_shape=_half_t,
    )(x, w)


def _tc_mid(a, b, w):
    return pl.pallas_call(
        _mid_body,
        grid=(N // TB,),
        in_specs=[_ablk,
                  pl.BlockSpec((1, D), lambda i: (0, 0)),
                  pl.BlockSpec((D, D), lambda i: (0, 0))],
        out_specs=_oblk,
        out_shape=_half_t,
    )(a, b, w)


def _tc_last(a, b):
    return pl.pallas_call(
        _last_body,
        grid=(N // TB,),
        in_specs=[_ablk, pl.BlockSpec((1, D), lambda i: (0, 0))],
        out_specs=(pl.BlockSpec((TB, D), lambda i: (i, 0)),
                   pl.BlockSpec((1, D), lambda i: (0, 0))),
        out_shape=(jax.ShapeDtypeStruct((N, D), jnp.float32),
                   jax.ShapeDtypeStruct((1, D), jnp.float32)),
    )(a, b)


def kernel(x, edge_index, w, W0, b0, W1, b1, W2, b2):
    src = edge_index[0]
    dst = edge_index[1]
    src2 = src.reshape(NS, NCHUNK, CHUNK)
    dst2 = dst.reshape(NS, NCHUNK, CHUNK)

    degp = _deg_kernel(src)
    inv = _tc_inv(degp).reshape(NPAD)
    c = _c_kernel(src, w, inv)

    t = _tc_mm(x, W0)
    a = _mp_kernel(t, src2, dst2, c)
    t = _tc_mid(a, b0.reshape(1, D), W1)
    a = _mp_kernel(t, src2, dst2, c)
    t = _tc_mid(a, b1.reshape(1, D), W2)
    a = _mp_kernel(t, src2, dst2, c)
    h, mN = _tc_last(a, b2.reshape(1, D))
    return (h, mN)


# speedup vs baseline: 1.6133x; 1.6133x over previous
import dataclasses
import functools

import jax
import jax.numpy as jnp
from jax import lax
from jax.experimental import pallas as pl
from jax.experimental.pallas import tpu as pltpu
from jax.experimental.pallas import tpu_sc as plsc

N = 10000
E = 320000
D = 128
DH = D // 2

NC = 2
NS = 16
NW = NC * NS
EPW = E // NW
EPT = E // NS
CHUNK = 100
NCHUNK = EPT // CHUNK
NBUF = 4
NPAD = 10240
L = 16
ZSTRIDE = 624
ZROWS = 640

_mesh = plsc.VectorSubcoreMesh(core_axis_name="c", subcore_axis_name="s")

_sc_params = pltpu.CompilerParams()
for _field, _val in (("needs_layout_passes", False),
                     ("use_tc_tiling_on_sc", False)):
    if _field in pltpu.CompilerParams.__dataclass_fields__:
        _sc_params = dataclasses.replace(_sc_params, **{_field: _val})


@functools.partial(
    pl.kernel,
    mesh=_mesh,
    compiler_params=_sc_params,
    out_type=jax.ShapeDtypeStruct((NW, NPAD), jnp.float32),
    scratch_types=[
        pltpu.VMEM((EPW,), jnp.int32),
        pltpu.VMEM((NPAD,), jnp.float32),
    ],
)
def _deg_kernel(src_hbm, degp_hbm, src_v, hist_v):
    cid = lax.axis_index("c")
    sid = lax.axis_index("s")
    wid = sid * NC + cid
    pltpu.sync_copy(src_hbm.at[pl.ds(wid * EPW, EPW)], src_v)

    zeros = jnp.zeros((L,), jnp.float32)
    ones = jnp.ones((L,), jnp.float32)

    @pl.loop(0, NPAD // L)
    def _(i):
        hist_v[pl.ds(i * L, L)] = zeros

    @pl.loop(0, EPW // L)
    def _(i):
        idx = src_v[pl.ds(i * L, L)]
        plsc.addupdate_scatter(hist_v, [idx], ones)

    pltpu.sync_copy(hist_v, degp_hbm.at[wid])


@functools.partial(
    pl.kernel,
    mesh=_mesh,
    compiler_params=_sc_params,
    out_type=jax.ShapeDtypeStruct((E,), jnp.float32),
    scratch_types=[
        pltpu.VMEM((EPW,), jnp.int32),
        pltpu.VMEM((EPW,), jnp.float32),
        pltpu.VMEM((NPAD,), jnp.float32),
    ],
)
def _c_kernel(src_hbm, w_hbm, inv_hbm, c_hbm, src_v, c_v, inv_v):
    cid = lax.axis_index("c")
    sid = lax.axis_index("s")
    wid = sid * NC + cid
    pltpu.sync_copy(src_hbm.at[pl.ds(wid * EPW, EPW)], src_v)
    pltpu.sync_copy(w_hbm.at[pl.ds(wid * EPW, EPW)], c_v)
    pltpu.sync_copy(inv_hbm, inv_v)

    @pl.loop(0, EPW // L)
    def _(i):
        s16 = src_v[pl.ds(i * L, L)]
        iv = plsc.load_gather(inv_v, [s16])
        c_v[pl.ds(i * L, L)] = c_v[pl.ds(i * L, L)] * iv

    pltpu.sync_copy(c_v, c_hbm.at[pl.ds(wid * EPW, EPW)])


@functools.partial(
    pl.kernel,
    mesh=_mesh,
    compiler_params=_sc_params,
    out_type=jax.ShapeDtypeStruct((NC, N, DH), jnp.float32),
    scratch_types=[
        pltpu.VMEM((NCHUNK, CHUNK), jnp.int32),
        pltpu.VMEM((NCHUNK, CHUNK), jnp.int32),
        pltpu.VMEM((EPT,), jnp.float32),
        pltpu.VMEM((NBUF, CHUNK, DH), jnp.float32),
        pltpu.SemaphoreType.DMA,
        pltpu.SemaphoreType.DMA,
        pltpu.SemaphoreType.DMA,
        pltpu.SemaphoreType.DMA,
        pltpu.SemaphoreType.DMA,
        pltpu.VMEM_SHARED((N, DH), jnp.float32),
    ],
)
def _mp_kernel(table_hbm, src2_hbm, dst2_hbm, c_hbm,
               acc_hbm, src2_v, dst2_v, c_v, gbuf_v,
               gsem0, gsem1, gsem2, gsem3, ssem, acc_sh):
    cid = lax.axis_index("c")
    sid = lax.axis_index("s")

    pltpu.sync_copy(src2_hbm.at[sid], src2_v)
    pltpu.sync_copy(dst2_hbm.at[sid], dst2_v)
    pltpu.sync_copy(c_hbm.at[pl.ds(sid * EPT, EPT)], c_v)

    zeros = jnp.zeros((L,), jnp.float32)

    @pl.loop(0, CHUNK)
    def _(e):
        for k in range(DH // L):
            gbuf_v[0, e, pl.ds(k * L, L)] = zeros

    base = sid * ZSTRIDE
    for r in range(ZROWS // CHUNK):
        pltpu.sync_copy(gbuf_v.at[0], acc_sh.at[pl.ds(base + r * CHUNK, CHUNK)])
    if ZROWS % CHUNK:
        pltpu.sync_copy(gbuf_v.at[0, pl.ds(0, ZROWS % CHUNK)],
                        acc_sh.at[pl.ds(base + (ZROWS // CHUNK) * CHUNK,
                                        ZROWS % CHUNK)])
    plsc.subcore_barrier()

    gsems = (gsem0, gsem1, gsem2, gsem3)

    def _gather(b, j):
        return pltpu.make_async_copy(table_hbm.at[cid].at[src2_v.at[j]],
                                     gbuf_v.at[b], gsems[b])

    def _scatter(b, j):
        return pltpu.async_copy(gbuf_v.at[b], acc_sh.at[dst2_v.at[j]],
                                ssem, add=True)

    def _scale(b, j):
        @pl.loop(0, CHUNK)
        def _(e):
            cvec = plsc.load_gather(
                c_v, [jnp.full((L,), j * CHUNK + e, jnp.int32)])
            for k in range(DH // L):
                gbuf_v[b, e, pl.ds(k * L, L)] = (
                    gbuf_v[b, e, pl.ds(k * L, L)] * cvec)

    for b in range(NBUF):
        _gather(b, b).start()

    @pl.loop(0, NCHUNK // NBUF)
    def _(t):
        for b in range(NBUF):
            j = NBUF * t + b
            _gather(b, j).wait()
            _scale(b, j)
            _scatter(b, j).wait()

            @pl.when(j + NBUF < NCHUNK)
            def _():
                _gather(b, j + NBUF).start()

    plsc.subcore_barrier()
    pltpu.sync_copy(acc_sh.at[pl.ds(base, ZROWS)],
                    acc_hbm.at[cid, pl.ds(base, ZROWS)])


def _inv_body(degp_ref, inv_ref):
    deg = jnp.sum(degp_ref[...], axis=0, keepdims=True)
    inv_ref[...] = 1.0 / jnp.maximum(deg, 1.0)


def _tc_inv(degp):
    return pl.pallas_call(
        _inv_body,
        out_shape=jax.ShapeDtypeStruct((1, NPAD), jnp.float32),
    )(degp)


def _mm_body(x_ref, w_ref, o_ref):
    x = x_ref[...]
    o_ref[0] = jnp.dot(x, w_ref[:, :DH], preferred_element_type=jnp.float32)
    o_ref[1] = jnp.dot(x, w_ref[:, DH:], preferred_element_type=jnp.float32)


def _mid_body(a_ref, b_ref, w_ref, o_ref):
    h = jnp.concatenate([a_ref[0], a_ref[1]], axis=1) + b_ref[...]
    h = jnp.maximum(h, 0.0)
    o_ref[0] = jnp.dot(h, w_ref[:, :DH], preferred_element_type=jnp.float32)
    o_ref[1] = jnp.dot(h, w_ref[:, DH:], preferred_element_type=jnp.float32)


def _last_body(a_ref, b_ref, h_ref, m_ref):
    h = jnp.concatenate([a_ref[0], a_ref[1]], axis=1) + b_ref[...]
    h_ref[...] = h
    m_ref[...] = jnp.sum(h, axis=0, keepdims=True)


_half_t = jax.ShapeDtypeStruct((NC, N, DH), jnp.float32)


def _tc_mm(x, w):
    return pl.pallas_call(_mm_body, out_shape=_half_t)(x, w)


def _tc_mid(a, b, w):
    return pl.pallas_call(_mid_body, out_shape=_half_t)(a, b, w)


def _tc_last(a, b):
    return pl.pallas_call(
        _last_body,
        out_shape=(jax.ShapeDtypeStruct((N, D), jnp.float32),
                   jax.ShapeDtypeStruct((1, D), jnp.float32)),
    )(a, b)


def kernel(x, edge_index, w, W0, b0, W1, b1, W2, b2):
    src = edge_index[0]
    dst = edge_index[1]
    src2 = src.reshape(NS, NCHUNK, CHUNK)
    dst2 = dst.reshape(NS, NCHUNK, CHUNK)

    degp = _deg_kernel(src)
    inv = _tc_inv(degp).reshape(NPAD)
    c = _c_kernel(src, w, inv)

    t = _tc_mm(x, W0)
    a = _mp_kernel(t, src2, dst2, c)
    t = _tc_mid(a, b0.reshape(1, D), W1)
    a = _mp_kernel(t, src2, dst2, c)
    t = _tc_mid(a, b1.reshape(1, D), W2)
    a = _mp_kernel(t, src2, dst2, c)
    h, mN = _tc_last(a, b2.reshape(1, D))
    return (h, mN)
